# baseline (speedup 1.0000x reference)
import functools

import jax
import jax.numpy as jnp
from jax import lax
from jax.experimental import pallas as pl
from jax.experimental.pallas import tpu as pltpu
from jax.experimental.pallas import tpu_sc as plsc

D = 128
DH = 64
L = 16
NC = 2
NS = 16
K = 128
NB = 4
RB = 512
N_PAD = 10240


def _dense_tail(i, xin, w_ref, as_ref, ad_ref, h_ref, a_s_ref, a_d_ref,
                shift_ref, mx_ref, n_real):
    h = jnp.dot(xin, w_ref[...], preferred_element_type=jnp.float32)
    h_ref[...] = h
    a_s = jnp.sum(h * as_ref[...], axis=1, keepdims=True)
    a_d = jnp.sum(h * ad_ref[...], axis=1, keepdims=True)
    rows = i * RB + lax.broadcasted_iota(jnp.int32, (RB, 1), 0)
    valid = rows < n_real
    neg_inf = jnp.float32(-jnp.inf)
    a_s = jnp.where(valid, a_s, neg_inf)
    a_d = jnp.where(valid, a_d, neg_inf)
    a_s_ref[...] = a_s
    a_d_ref[...] = a_d
    bs = jnp.max(a_s)
    bd = jnp.max(a_d)

    @pl.when(i == 0)
    def _():
        mx_ref[0] = bs
        mx_ref[1] = bd

    @pl.when(i > 0)
    def _():
        mx_ref[0] = jnp.maximum(mx_ref[0], bs)
        mx_ref[1] = jnp.maximum(mx_ref[1], bd)

    shift_ref[...] = jnp.full(
        (8, 128), jnp.maximum(mx_ref[0] + mx_ref[1], 0.0), jnp.float32)


def _prep_body(x_ref, w_ref, as_ref, ad_ref,
               h_ref, a_s_ref, a_d_ref, shift_ref, mx_ref, *, n_real):
    i = pl.program_id(0)
    _dense_tail(i, x_ref[...], w_ref, as_ref, ad_ref,
                h_ref, a_s_ref, a_d_ref, shift_ref, mx_ref, n_real)


def _gat_out_block(i, n_ref, d_ref, b_ref, n_real):
    numer = jnp.concatenate([n_ref[0], n_ref[1]], axis=-1)
    den = d_ref[..., 0:1]
    xin = numer / (den + 1e-16) + b_ref[...]
    xin = jnp.where(xin > 0, xin, jnp.exp(jnp.minimum(xin, 0.0)) - 1.0)
    rows = i * RB + lax.broadcasted_iota(jnp.int32, (RB, 1), 0)
    return jnp.where(rows < n_real, xin, 0.0)


def _mid_body(n_ref, d_ref, b_ref, w_ref, as_ref, ad_ref,
              h_ref, a_s_ref, a_d_ref, shift_ref, mx_ref, *, n_real):
    i = pl.program_id(0)
    xin = _gat_out_block(i, n_ref, d_ref, b_ref, n_real)
    _dense_tail(i, xin, w_ref, as_ref, ad_ref,
                h_ref, a_s_ref, a_d_ref, shift_ref, mx_ref, n_real)


def _final_body(n_ref, d_ref, b_ref, w_ref, bl_ref, out_ref, *, n_real):
    i = pl.program_id(0)
    xin = _gat_out_block(i, n_ref, d_ref, b_ref, n_real)
    out_ref[...] = (
        jnp.dot(xin, w_ref[...], preferred_element_type=jnp.float32)
        + bl_ref[...])


_DENSE_OUT = [
    jax.ShapeDtypeStruct((N_PAD, D), jnp.float32),
    jax.ShapeDtypeStruct((N_PAD, 1), jnp.float32),
    jax.ShapeDtypeStruct((N_PAD, 1), jnp.float32),
    jax.ShapeDtypeStruct((8, 128), jnp.float32),
]
_DENSE_OUT_SPECS = [
    pl.BlockSpec((RB, D), lambda i: (i, 0)),
    pl.BlockSpec((RB, 1), lambda i: (i, 0)),
    pl.BlockSpec((RB, 1), lambda i: (i, 0)),
    pl.BlockSpec((8, 128), lambda i: (0, 0)),
]


def _tc_prep(xp, w, att_s, att_d, n_real):
    grid = N_PAD // RB
    return pl.pallas_call(
        functools.partial(_prep_body, n_real=n_real),
        grid=(grid,),
        in_specs=[
            pl.BlockSpec((RB, D), lambda i: (i, 0)),
            pl.BlockSpec((D, D), lambda i: (0, 0)),
            pl.BlockSpec((1, D), lambda i: (0, 0)),
            pl.BlockSpec((1, D), lambda i: (0, 0)),
        ],
        out_specs=_DENSE_OUT_SPECS,
        out_shape=_DENSE_OUT,
        scratch_shapes=[pltpu.SMEM((2,), jnp.float32)],
    )(xp, w, att_s, att_d)


def _tc_mid(num, den0, b, w, att_s, att_d, n_real):
    grid = N_PAD // RB
    return pl.pallas_call(
        functools.partial(_mid_body, n_real=n_real),
        grid=(grid,),
        in_specs=[
            pl.BlockSpec((NC, RB, DH), lambda i: (0, i, 0)),
            pl.BlockSpec((RB, L), lambda i: (i, 0)),
            pl.BlockSpec((1, D), lambda i: (0, 0)),
            pl.BlockSpec((D, D), lambda i: (0, 0)),
            pl.BlockSpec((1, D), lambda i: (0, 0)),
            pl.BlockSpec((1, D), lambda i: (0, 0)),
        ],
        out_specs=_DENSE_OUT_SPECS,
        out_shape=_DENSE_OUT,
        scratch_shapes=[pltpu.SMEM((2,), jnp.float32)],
    )(num, den0, b, w, att_s, att_d)


def _tc_final(num, den0, b, w, bl, n_real):
    grid = N_PAD // RB
    return pl.pallas_call(
        functools.partial(_final_body, n_real=n_real),
        grid=(grid,),
        in_specs=[
            pl.BlockSpec((NC, RB, DH), lambda i: (0, i, 0)),
            pl.BlockSpec((RB, L), lambda i: (i, 0)),
            pl.BlockSpec((1, D), lambda i: (0, 0)),
            pl.BlockSpec((D, D), lambda i: (0, 0)),
            pl.BlockSpec((1, D), lambda i: (0, 0)),
        ],
        out_specs=pl.BlockSpec((RB, D), lambda i: (i, 0)),
        out_shape=jax.ShapeDtypeStruct((N_PAD, D), jnp.float32),
    )(num, den0, b, w, bl)


def _make_sc_edge(chunks):
    mesh = plsc.VectorSubcoreMesh(core_axis_name="c", subcore_axis_name="s")
    rows_per_sub = N_PAD // NS

    @functools.partial(
        pl.kernel,
        out_type=(
            jax.ShapeDtypeStruct((NC, N_PAD, DH), jnp.float32),
            jax.ShapeDtypeStruct((NC, N_PAD, L), jnp.float32),
        ),
        mesh=mesh,
        compiler_params=pltpu.CompilerParams(
            needs_layout_passes=False, use_tc_tiling_on_sc=False,
            skip_device_barrier=True),
        scratch_types=[
            pltpu.VMEM_SHARED((N_PAD, DH), jnp.float32),
            pltpu.VMEM_SHARED((N_PAD, L), jnp.float32),
            pltpu.VMEM((N_PAD // D, D), jnp.float32),
            pltpu.VMEM((N_PAD // D, D), jnp.float32),
            pltpu.VMEM((NB, K), jnp.int32),
            pltpu.VMEM((NB, K), jnp.int32),
            pltpu.VMEM((NB, K, DH), jnp.float32),
            pltpu.VMEM((NB, K, L), jnp.float32),
            pltpu.VMEM((NB, K), jnp.float32),
            pltpu.VMEM((L,), jnp.float32),
            pltpu.SemaphoreType.DMA((NB,)),
            pltpu.SemaphoreType.DMA((NB,)),
            pltpu.SemaphoreType.DMA((NB,)),
            pltpu.SemaphoreType.DMA((NB,)),
        ],
    )
    def sc_edge(h_hbm, asrc_hbm, adst_hbm, src_hbm, dst_hbm, shift_hbm,
                numer_hbm, den_hbm,
                nacc, dacc, asrc_v, adst_v, src_v, dst_v,
                raw_v, wrow_v, w_v, shift_v, sem_i, sem_g, sem_s, sem_d):
        c = lax.axis_index("c")
        s = lax.axis_index("s")

        def zrow(r, carry):
            for j in range(DH // L):
                raw_v[0, r, pl.ds(j * L, L)] = jnp.zeros((L,), jnp.float32)
            for b in range(NB):
                wrow_v[b, r, pl.ds(0, L)] = jnp.zeros((L,), jnp.float32)
            return carry
        lax.fori_loop(0, K, zrow, 0)

        def zacc(t, carry):
            base = s * rows_per_sub + t * K
            pltpu.sync_copy(raw_v.at[0], nacc.at[pl.ds(base, K)])
            pltpu.sync_copy(wrow_v.at[0], dacc.at[pl.ds(base, K)])
            return carry
        lax.fori_loop(0, rows_per_sub // K, zacc, 0)

        pltpu.sync_copy(asrc_hbm, asrc_v)
        pltpu.sync_copy(adst_hbm, adst_v)
        pltpu.sync_copy(shift_hbm, shift_v)

        plsc.subcore_barrier()

        lane = lax.iota(jnp.int32, L)
        zero_lane = jnp.zeros((L,), jnp.int32)
        low_mask = jnp.full((L,), D - 1, jnp.int32)

        def issue_idx(t, b):
            pltpu.async_copy(src_hbm.at[c, s, t], src_v.at[b], sem_i.at[b])
            pltpu.async_copy(dst_hbm.at[s, t], dst_v.at[b], sem_i.at[b])

        def wait_idx(t, b):
            pltpu.make_async_copy(src_hbm.at[c, s, t], src_v.at[b], sem_i.at[b]).wait()
            pltpu.make_async_copy(dst_hbm.at[s, t], dst_v.at[b], sem_i.at[b]).wait()

        def issue_gather(b):
            pltpu.async_copy(h_hbm.at[src_v.at[b]], raw_v.at[b], sem_g.at[b])

        def wait_gather(b):
            pltpu.make_async_copy(h_hbm.at[src_v.at[b]], raw_v.at[b],
                                  sem_g.at[b]).wait()

        def drain_scatter(b):
            pltpu.make_async_copy(raw_v.at[b], nacc.at[dst_v.at[b]],
                                  sem_s.at[b]).wait()
            pltpu.make_async_copy(wrow_v.at[b], dacc.at[dst_v.at[b]],
                                  sem_d.at[b]).wait()

        issue_idx(0, 0)
        issue_idx(1, 1)
        wait_idx(0, 0)
        issue_gather(0)

        def chunk_body(t, carry):
            @pl.when(t + 2 < chunks)
            def _():
                b2 = (t + 2) % NB

                @pl.when(t + 2 >= NB)
                def _():
                    drain_scatter((t + 2) % NB)
                issue_idx(t + 2, (t + 2) % NB)

            @pl.when(t + 1 < chunks)
            def _():
                b1 = (t + 1) % NB
                wait_idx(t + 1, b1)
                issue_gather(b1)

            b = t % NB

            def wgrp(g, carry2):
                si = src_v[b, pl.ds(g * L, L)]
                di = dst_v[b, pl.ds(g * L, L)]
                a_s = plsc.load_gather(
                    asrc_v,
                    [lax.shift_right_logical(si, 8),
                     lax.bitwise_and(lax.shift_right_logical(si, 1), low_mask)])
                a_d = plsc.load_gather(
                    adst_v,
                    [lax.shift_right_logical(di, 7), lax.bitwise_and(di, low_mask)])
                e = a_s + a_d
                lr = jnp.where(e >= 0.0, e, e * jnp.float32(0.2))
                w = jnp.exp(lr - shift_v[...])
                w_v[b, pl.ds(g * L, L)] = w
                plsc.store_scatter(wrow_v.at[b], [g * L + lane, zero_lane], w)
                return carry2
            lax.fori_loop(0, K // L, wgrp, 0)

            wait_gather(b)

            def erow(g, carry2):
                wv = w_v[b, pl.ds(g * L, L)]
                for ee in range(L):
                    r = g * L + ee
                    ws = wv[ee]
                    for j in range(DH // L):
                        raw_v[b, r, pl.ds(j * L, L)] = (
                            raw_v[b, r, pl.ds(j * L, L)] * ws)
                return carry2
            lax.fori_loop(0, K // L, erow, 0)

            pltpu.async_copy(raw_v.at[b], nacc.at[dst_v.at[b]], sem_s.at[b],
                             add=True)
            pltpu.async_copy(wrow_v.at[b], dacc.at[dst_v.at[b]], sem_d.at[b],
                             add=True)
            return carry
        lax.fori_loop(0, chunks, chunk_body, 0)

        for b in range(NB):
            drain_scatter(b)

        plsc.subcore_barrier()

        base = s * rows_per_sub
        pltpu.sync_copy(nacc.at[pl.ds(base, rows_per_sub)],
                        numer_hbm.at[c, pl.ds(base, rows_per_sub), :])
        pltpu.sync_copy(dacc.at[pl.ds(base, rows_per_sub)],
                        den_hbm.at[c, pl.ds(base, rows_per_sub), :])

    return sc_edge


def kernel(x, edge_index, W1, att_src1, att_dst1, b1,
           W2, att_src2, att_dst2, b2, W_lin, b_lin):
    n_real, d_in = x.shape
    n_cls = W_lin.shape[1]
    e_raw = edge_index.shape[1] + n_real
    e_pad = ((e_raw + NS * K - 1) // (NS * K)) * (NS * K)
    chunks = e_pad // (NS * K)

    loops = jnp.arange(n_real, dtype=jnp.int32)
    pad = jnp.full((e_pad - e_raw,), n_real, jnp.int32)
    src1 = jnp.concatenate([edge_index[0], loops, pad])
    src4 = jnp.stack([2 * src1, 2 * src1 + 1]).reshape(NC, NS, chunks, K)
    dst3 = jnp.concatenate([edge_index[1], loops, pad]).reshape(NS, chunks, K)

    xp = jnp.zeros((N_PAD, d_in), jnp.float32).at[:n_real].set(x)

    sc_edge = _make_sc_edge(chunks)

    h1, as1, ad1, shift1 = _tc_prep(
        xp, W1, att_src1.reshape(1, D), att_dst1.reshape(1, D), n_real)
    num1, den1 = sc_edge(h1.reshape(NC * N_PAD, DH),
                         as1.reshape(N_PAD // D, D), ad1.reshape(N_PAD // D, D),
                         src4, dst3, shift1[0, :L])

    h2, as2, ad2, shift2 = _tc_mid(
        num1, den1[0], b1.reshape(1, D), W2,
        att_src2.reshape(1, D), att_dst2.reshape(1, D), n_real)
    num2, den2 = sc_edge(h2.reshape(NC * N_PAD, DH),
                         as2.reshape(N_PAD // D, D), ad2.reshape(N_PAD // D, D),
                         src4, dst3, shift2[0, :L])

    wl = jnp.zeros((D, 128), jnp.float32).at[:, :n_cls].set(W_lin)
    bl = jnp.zeros((1, 128), jnp.float32).at[0, :n_cls].set(b_lin)
    out = _tc_final(num2, den2[0], b2.reshape(1, D), wl, bl, n_real)
    return out[:n_real, :n_cls]


# speedup vs baseline: 1.0092x; 1.0092x over previous
import functools

import jax
import jax.numpy as jnp
from jax import lax
from jax.experimental import pallas as pl
from jax.experimental.pallas import tpu as pltpu
from jax.experimental.pallas import tpu_sc as plsc

D = 128
DH = 64
L = 16
NC = 2
NS = 16
K = 128
NB = 4
RB = 256
N_PAD = 10240


def _dense_tail(i, xin, w_ref, as_ref, ad_ref, h_ref, a_s_ref, a_d_ref,
                shift_ref, mx_ref, n_real):
    h = jnp.dot(xin, w_ref[...], preferred_element_type=jnp.float32)
    h_ref[...] = jnp.stack([h[:, :DH], h[:, DH:]])
    a_s = jnp.sum(h * as_ref[...], axis=1, keepdims=True)
    a_d = jnp.sum(h * ad_ref[...], axis=1, keepdims=True)
    rows = i * RB + lax.broadcasted_iota(jnp.int32, (RB, 1), 0)
    valid = rows < n_real
    neg_inf = jnp.float32(-jnp.inf)
    a_s = jnp.where(valid, a_s, neg_inf)
    a_d = jnp.where(valid, a_d, neg_inf)
    a_s_ref[...] = a_s
    a_d_ref[...] = a_d
    bs = jnp.max(a_s)
    bd = jnp.max(a_d)

    @pl.when(i == 0)
    def _():
        mx_ref[0] = bs
        mx_ref[1] = bd

    @pl.when(i > 0)
    def _():
        mx_ref[0] = jnp.maximum(mx_ref[0], bs)
        mx_ref[1] = jnp.maximum(mx_ref[1], bd)

    shift_ref[...] = jnp.full(
        (8, 128), jnp.maximum(mx_ref[0] + mx_ref[1], 0.0), jnp.float32)


def _prep_body(x_ref, w_ref, as_ref, ad_ref,
               h_ref, a_s_ref, a_d_ref, shift_ref, mx_ref, *, n_real):
    i = pl.program_id(0)
    _dense_tail(i, x_ref[...], w_ref, as_ref, ad_ref,
                h_ref, a_s_ref, a_d_ref, shift_ref, mx_ref, n_real)


def _gat_out_block(i, n_ref, d_ref, b_ref, n_real):
    numer = jnp.concatenate([n_ref[0], n_ref[1]], axis=-1)
    den = d_ref[..., 0:1]
    xin = numer / (den + 1e-16) + b_ref[...]
    xin = jnp.where(xin > 0, xin, jnp.exp(jnp.minimum(xin, 0.0)) - 1.0)
    rows = i * RB + lax.broadcasted_iota(jnp.int32, (RB, 1), 0)
    return jnp.where(rows < n_real, xin, 0.0)


def _mid_body(n_ref, d_ref, b_ref, w_ref, as_ref, ad_ref,
              h_ref, a_s_ref, a_d_ref, shift_ref, mx_ref, *, n_real):
    i = pl.program_id(0)
    xin = _gat_out_block(i, n_ref, d_ref, b_ref, n_real)
    _dense_tail(i, xin, w_ref, as_ref, ad_ref,
                h_ref, a_s_ref, a_d_ref, shift_ref, mx_ref, n_real)


def _final_body(n_ref, d_ref, b_ref, w_ref, bl_ref, out_ref, *, n_real):
    i = pl.program_id(0)
    xin = _gat_out_block(i, n_ref, d_ref, b_ref, n_real)
    out_ref[...] = (
        jnp.dot(xin, w_ref[...], preferred_element_type=jnp.float32)
        + bl_ref[...])


_DENSE_OUT = [
    jax.ShapeDtypeStruct((NC, N_PAD, DH), jnp.float32),
    jax.ShapeDtypeStruct((N_PAD, 1), jnp.float32),
    jax.ShapeDtypeStruct((N_PAD, 1), jnp.float32),
    jax.ShapeDtypeStruct((8, 128), jnp.float32),
]
_DENSE_OUT_SPECS = [
    pl.BlockSpec((NC, RB, DH), lambda i: (0, i, 0)),
    pl.BlockSpec((RB, 1), lambda i: (i, 0)),
    pl.BlockSpec((RB, 1), lambda i: (i, 0)),
    pl.BlockSpec((8, 128), lambda i: (0, 0)),
]


def _tc_prep(xp, w, att_s, att_d, n_real):
    grid = N_PAD // RB
    return pl.pallas_call(
        functools.partial(_prep_body, n_real=n_real),
        grid=(grid,),
        in_specs=[
            pl.BlockSpec((RB, D), lambda i: (i, 0)),
            pl.BlockSpec((D, D), lambda i: (0, 0)),
            pl.BlockSpec((1, D), lambda i: (0, 0)),
            pl.BlockSpec((1, D), lambda i: (0, 0)),
        ],
        out_specs=_DENSE_OUT_SPECS,
        out_shape=_DENSE_OUT,
        scratch_shapes=[pltpu.SMEM((2,), jnp.float32)],
    )(xp, w, att_s, att_d)


def _tc_mid(num, den0, b, w, att_s, att_d, n_real):
    grid = N_PAD // RB
    return pl.pallas_call(
        functools.partial(_mid_body, n_real=n_real),
        grid=(grid,),
        in_specs=[
            pl.BlockSpec((NC, RB, DH), lambda i: (0, i, 0)),
            pl.BlockSpec((RB, L), lambda i: (i, 0)),
            pl.BlockSpec((1, D), lambda i: (0, 0)),
            pl.BlockSpec((D, D), lambda i: (0, 0)),
            pl.BlockSpec((1, D), lambda i: (0, 0)),
            pl.BlockSpec((1, D), lambda i: (0, 0)),
        ],
        out_specs=_DENSE_OUT_SPECS,
        out_shape=_DENSE_OUT,
        scratch_shapes=[pltpu.SMEM((2,), jnp.float32)],
    )(num, den0, b, w, att_s, att_d)


def _tc_final(num, den0, b, w, bl, n_real):
    grid = N_PAD // RB
    return pl.pallas_call(
        functools.partial(_final_body, n_real=n_real),
        grid=(grid,),
        in_specs=[
            pl.BlockSpec((NC, RB, DH), lambda i: (0, i, 0)),
            pl.BlockSpec((RB, L), lambda i: (i, 0)),
            pl.BlockSpec((1, D), lambda i: (0, 0)),
            pl.BlockSpec((D, D), lambda i: (0, 0)),
            pl.BlockSpec((1, D), lambda i: (0, 0)),
        ],
        out_specs=pl.BlockSpec((RB, D), lambda i: (i, 0)),
        out_shape=jax.ShapeDtypeStruct((N_PAD, D), jnp.float32),
    )(num, den0, b, w, bl)


def _make_sc_edge(chunks):
    mesh = plsc.VectorSubcoreMesh(core_axis_name="c", subcore_axis_name="s")
    rows_per_sub = N_PAD // NS

    @functools.partial(
        pl.kernel,
        out_type=(
            jax.ShapeDtypeStruct((NC, N_PAD, DH), jnp.float32),
            jax.ShapeDtypeStruct((NC, N_PAD, L), jnp.float32),
        ),
        mesh=mesh,
        compiler_params=pltpu.CompilerParams(
            needs_layout_passes=False, use_tc_tiling_on_sc=False),
        scratch_types=[
            pltpu.VMEM_SHARED((N_PAD, DH), jnp.float32),
            pltpu.VMEM_SHARED((N_PAD, L), jnp.float32),
            pltpu.VMEM((N_PAD // D, D), jnp.float32),
            pltpu.VMEM((N_PAD // D, D), jnp.float32),
            pltpu.VMEM((NB, K), jnp.int32),
            pltpu.VMEM((NB, K), jnp.int32),
            pltpu.VMEM((NB, K, DH), jnp.float32),
            pltpu.VMEM((NB, K, L), jnp.float32),
            pltpu.VMEM((NB, K), jnp.float32),
            pltpu.VMEM((L,), jnp.float32),
            pltpu.SemaphoreType.DMA((NB,)),
            pltpu.SemaphoreType.DMA((NB,)),
            pltpu.SemaphoreType.DMA((NB,)),
            pltpu.SemaphoreType.DMA((NB,)),
        ],
    )
    def sc_edge(h_hbm, asrc_hbm, adst_hbm, src_hbm, dst_hbm, shift_hbm,
                numer_hbm, den_hbm,
                nacc, dacc, asrc_v, adst_v, src_v, dst_v,
                raw_v, wrow_v, w_v, shift_v, sem_i, sem_g, sem_s, sem_d):
        c = lax.axis_index("c")
        s = lax.axis_index("s")

        def zrow(r, carry):
            for j in range(DH // L):
                raw_v[0, r, pl.ds(j * L, L)] = jnp.zeros((L,), jnp.float32)
            for b in range(NB):
                wrow_v[b, r, pl.ds(0, L)] = jnp.zeros((L,), jnp.float32)
            return carry
        lax.fori_loop(0, K, zrow, 0)

        def zacc(t, carry):
            base = s * rows_per_sub + t * K
            pltpu.sync_copy(raw_v.at[0], nacc.at[pl.ds(base, K)])
            pltpu.sync_copy(wrow_v.at[0], dacc.at[pl.ds(base, K)])
            return carry
        lax.fori_loop(0, rows_per_sub // K, zacc, 0)

        pltpu.sync_copy(asrc_hbm, asrc_v)
        pltpu.sync_copy(adst_hbm, adst_v)
        pltpu.sync_copy(shift_hbm, shift_v)

        plsc.subcore_barrier()

        lane = lax.iota(jnp.int32, L)
        zero_lane = jnp.zeros((L,), jnp.int32)
        low_mask = jnp.full((L,), D - 1, jnp.int32)

        def issue_idx(t, b):
            pltpu.async_copy(src_hbm.at[s, t], src_v.at[b], sem_i.at[b])
            pltpu.async_copy(dst_hbm.at[s, t], dst_v.at[b], sem_i.at[b])

        def wait_idx(t, b):
            pltpu.make_async_copy(src_hbm.at[s, t], src_v.at[b], sem_i.at[b]).wait()
            pltpu.make_async_copy(dst_hbm.at[s, t], dst_v.at[b], sem_i.at[b]).wait()

        def issue_gather(b):
            pltpu.async_copy(h_hbm.at[c].at[src_v.at[b]], raw_v.at[b],
                             sem_g.at[b])

        def wait_gather(b):
            pltpu.make_async_copy(h_hbm.at[c].at[src_v.at[b]], raw_v.at[b],
                                  sem_g.at[b]).wait()

        def drain_scatter(b):
            pltpu.make_async_copy(raw_v.at[b], nacc.at[dst_v.at[b]],
                                  sem_s.at[b]).wait()
            pltpu.make_async_copy(wrow_v.at[b], dacc.at[dst_v.at[b]],
                                  sem_d.at[b]).wait()

        issue_idx(0, 0)
        issue_idx(1, 1)
        wait_idx(0, 0)
        issue_gather(0)

        def chunk_body(t, carry):
            @pl.when(t + 2 < chunks)
            def _():
                b2 = (t + 2) % NB

                @pl.when(t + 2 >= NB)
                def _():
                    drain_scatter((t + 2) % NB)
                issue_idx(t + 2, (t + 2) % NB)

            @pl.when(t + 1 < chunks)
            def _():
                b1 = (t + 1) % NB
                wait_idx(t + 1, b1)
                issue_gather(b1)

            b = t % NB

            def wgrp(g, carry2):
                si = src_v[b, pl.ds(g * L, L)]
                di = dst_v[b, pl.ds(g * L, L)]
                a_s = plsc.load_gather(
                    asrc_v,
                    [lax.shift_right_logical(si, 7), lax.bitwise_and(si, low_mask)])
                a_d = plsc.load_gather(
                    adst_v,
                    [lax.shift_right_logical(di, 7), lax.bitwise_and(di, low_mask)])
                e = a_s + a_d
                lr = jnp.where(e >= 0.0, e, e * jnp.float32(0.2))
                w = jnp.exp(lr - shift_v[...])
                w_v[b, pl.ds(g * L, L)] = w
                plsc.store_scatter(wrow_v.at[b], [g * L + lane, zero_lane], w)
                return carry2
            lax.fori_loop(0, K // L, wgrp, 0)

            wait_gather(b)

            def erow(g, carry2):
                wv = w_v[b, pl.ds(g * L, L)]
                for ee in range(L):
                    r = g * L + ee
                    ws = wv[ee]
                    for j in range(DH // L):
                        raw_v[b, r, pl.ds(j * L, L)] = (
                            raw_v[b, r, pl.ds(j * L, L)] * ws)
                return carry2
            lax.fori_loop(0, K // L, erow, 0)

            pltpu.async_copy(raw_v.at[b], nacc.at[dst_v.at[b]], sem_s.at[b],
                             add=True)
            pltpu.async_copy(wrow_v.at[b], dacc.at[dst_v.at[b]], sem_d.at[b],
                             add=True)
            return carry
        lax.fori_loop(0, chunks, chunk_body, 0, unroll=2)

        for b in range(NB):
            drain_scatter(b)

        plsc.subcore_barrier()

        base = s * rows_per_sub
        pltpu.sync_copy(nacc.at[pl.ds(base, rows_per_sub)],
                        numer_hbm.at[c, pl.ds(base, rows_per_sub), :])
        pltpu.sync_copy(dacc.at[pl.ds(base, rows_per_sub)],
                        den_hbm.at[c, pl.ds(base, rows_per_sub), :])

    return sc_edge


def kernel(x, edge_index, W1, att_src1, att_dst1, b1,
           W2, att_src2, att_dst2, b2, W_lin, b_lin):
    n_real, d_in = x.shape
    n_cls = W_lin.shape[1]
    e_raw = edge_index.shape[1] + n_real
    e_pad = ((e_raw + NS * K - 1) // (NS * K)) * (NS * K)
    chunks = e_pad // (NS * K)

    loops = jnp.arange(n_real, dtype=jnp.int32)
    pad = jnp.full((e_pad - e_raw,), n_real, jnp.int32)
    src3 = jnp.concatenate([edge_index[0], loops, pad]).reshape(NS, chunks, K)
    dst3 = jnp.concatenate([edge_index[1], loops, pad]).reshape(NS, chunks, K)

    xp = jnp.zeros((N_PAD, d_in), jnp.float32).at[:n_real].set(x)

    sc_edge = _make_sc_edge(chunks)

    h1, as1, ad1, shift1 = _tc_prep(
        xp, W1, att_src1.reshape(1, D), att_dst1.reshape(1, D), n_real)
    num1, den1 = sc_edge(h1, as1.reshape(N_PAD // D, D), ad1.reshape(N_PAD // D, D),
                         src3, dst3, shift1[0, :L])

    h2, as2, ad2, shift2 = _tc_mid(
        num1, den1[0], b1.reshape(1, D), W2,
        att_src2.reshape(1, D), att_dst2.reshape(1, D), n_real)
    num2, den2 = sc_edge(h2, as2.reshape(N_PAD // D, D), ad2.reshape(N_PAD // D, D),
                         src3, dst3, shift2[0, :L])

    wl = jnp.zeros((D, 128), jnp.float32).at[:, :n_cls].set(W_lin)
    bl = jnp.zeros((1, 128), jnp.float32).at[0, :n_cls].set(b_lin)
    out = _tc_final(num2, den2[0], b2.reshape(1, D), wl, bl, n_real)
    return out[:n_real, :n_cls]


# speedup vs baseline: 1.0193x; 1.0100x over previous
import functools

import jax
import jax.numpy as jnp
from jax import lax
from jax.experimental import pallas as pl
from jax.experimental.pallas import tpu as pltpu
from jax.experimental.pallas import tpu_sc as plsc

D = 128
DH = 64
L = 16
NC = 2
NS = 16
K = 128
NB = 4
RB = 256
N_PAD = 10240


def _dense_tail(i, xin, w_ref, as_ref, ad_ref, h_ref, a_s_ref, a_d_ref,
                shift_ref, mx_ref, n_real):
    h = jnp.dot(xin, w_ref[...], preferred_element_type=jnp.float32)
    h_ref[...] = jnp.stack([h[:, :DH], h[:, DH:]])
    a_s = jnp.sum(h * as_ref[...], axis=1, keepdims=True)
    a_d = jnp.sum(h * ad_ref[...], axis=1, keepdims=True)
    rows = i * RB + lax.broadcasted_iota(jnp.int32, (RB, 1), 0)
    valid = rows < n_real
    neg_inf = jnp.float32(-jnp.inf)
    a_s = jnp.where(valid, a_s, neg_inf)
    a_d = jnp.where(valid, a_d, neg_inf)
    a_s_ref[...] = a_s
    a_d_ref[...] = a_d
    bs = jnp.max(a_s)
    bd = jnp.max(a_d)

    @pl.when(i == 0)
    def _():
        mx_ref[0] = bs
        mx_ref[1] = bd

    @pl.when(i > 0)
    def _():
        mx_ref[0] = jnp.maximum(mx_ref[0], bs)
        mx_ref[1] = jnp.maximum(mx_ref[1], bd)

    shift_ref[...] = jnp.full(
        (8, 128), jnp.maximum(mx_ref[0] + mx_ref[1], 0.0), jnp.float32)


def _prep_body(x_ref, w_ref, as_ref, ad_ref,
               h_ref, a_s_ref, a_d_ref, shift_ref, mx_ref, *, n_real):
    i = pl.program_id(0)
    _dense_tail(i, x_ref[...], w_ref, as_ref, ad_ref,
                h_ref, a_s_ref, a_d_ref, shift_ref, mx_ref, n_real)


def _gat_out_block(i, n_ref, d_ref, b_ref, n_real):
    numer = jnp.concatenate([n_ref[0], n_ref[1]], axis=-1)
    den = d_ref[0, :, 0:1] + d_ref[1, :, 0:1]
    xin = numer / (den + 1e-16) + b_ref[...]
    xin = jnp.where(xin > 0, xin, jnp.exp(jnp.minimum(xin, 0.0)) - 1.0)
    rows = i * RB + lax.broadcasted_iota(jnp.int32, (RB, 1), 0)
    return jnp.where(rows < n_real, xin, 0.0)


def _mid_body(n_ref, d_ref, b_ref, w_ref, as_ref, ad_ref,
              h_ref, a_s_ref, a_d_ref, shift_ref, mx_ref, *, n_real):
    i = pl.program_id(0)
    xin = _gat_out_block(i, n_ref, d_ref, b_ref, n_real)
    _dense_tail(i, xin, w_ref, as_ref, ad_ref,
                h_ref, a_s_ref, a_d_ref, shift_ref, mx_ref, n_real)


def _final_body(n_ref, d_ref, b_ref, w_ref, bl_ref, out_ref, *, n_real):
    i = pl.program_id(0)
    xin = _gat_out_block(i, n_ref, d_ref, b_ref, n_real)
    out_ref[...] = (
        jnp.dot(xin, w_ref[...], preferred_element_type=jnp.float32)
        + bl_ref[...])


_DENSE_OUT = [
    jax.ShapeDtypeStruct((NC, N_PAD, DH), jnp.float32),
    jax.ShapeDtypeStruct((N_PAD, 1), jnp.float32),
    jax.ShapeDtypeStruct((N_PAD, 1), jnp.float32),
    jax.ShapeDtypeStruct((8, 128), jnp.float32),
]
_DENSE_OUT_SPECS = [
    pl.BlockSpec((NC, RB, DH), lambda i: (0, i, 0)),
    pl.BlockSpec((RB, 1), lambda i: (i, 0)),
    pl.BlockSpec((RB, 1), lambda i: (i, 0)),
    pl.BlockSpec((8, 128), lambda i: (0, 0)),
]


def _tc_prep(xp, w, att_s, att_d, n_real):
    grid = N_PAD // RB
    return pl.pallas_call(
        functools.partial(_prep_body, n_real=n_real),
        grid=(grid,),
        in_specs=[
            pl.BlockSpec((RB, D), lambda i: (i, 0)),
            pl.BlockSpec((D, D), lambda i: (0, 0)),
            pl.BlockSpec((1, D), lambda i: (0, 0)),
            pl.BlockSpec((1, D), lambda i: (0, 0)),
        ],
        out_specs=_DENSE_OUT_SPECS,
        out_shape=_DENSE_OUT,
        scratch_shapes=[pltpu.SMEM((2,), jnp.float32)],
    )(xp, w, att_s, att_d)


def _tc_mid(num, den0, b, w, att_s, att_d, n_real):
    grid = N_PAD // RB
    return pl.pallas_call(
        functools.partial(_mid_body, n_real=n_real),
        grid=(grid,),
        in_specs=[
            pl.BlockSpec((NC, RB, DH), lambda i: (0, i, 0)),
            pl.BlockSpec((NC, RB, L), lambda i: (0, i, 0)),
            pl.BlockSpec((1, D), lambda i: (0, 0)),
            pl.BlockSpec((D, D), lambda i: (0, 0)),
            pl.BlockSpec((1, D), lambda i: (0, 0)),
            pl.BlockSpec((1, D), lambda i: (0, 0)),
        ],
        out_specs=_DENSE_OUT_SPECS,
        out_shape=_DENSE_OUT,
        scratch_shapes=[pltpu.SMEM((2,), jnp.float32)],
    )(num, den0, b, w, att_s, att_d)


def _tc_final(num, den0, b, w, bl, n_real):
    grid = N_PAD // RB
    return pl.pallas_call(
        functools.partial(_final_body, n_real=n_real),
        grid=(grid,),
        in_specs=[
            pl.BlockSpec((NC, RB, DH), lambda i: (0, i, 0)),
            pl.BlockSpec((NC, RB, L), lambda i: (0, i, 0)),
            pl.BlockSpec((1, D), lambda i: (0, 0)),
            pl.BlockSpec((D, D), lambda i: (0, 0)),
            pl.BlockSpec((1, D), lambda i: (0, 0)),
        ],
        out_specs=pl.BlockSpec((RB, D), lambda i: (i, 0)),
        out_shape=jax.ShapeDtypeStruct((N_PAD, D), jnp.float32),
    )(num, den0, b, w, bl)


def _make_sc_edge(chunks):
    mesh = plsc.VectorSubcoreMesh(core_axis_name="c", subcore_axis_name="s")
    rows_per_sub = N_PAD // NS

    @functools.partial(
        pl.kernel,
        out_type=(
            jax.ShapeDtypeStruct((NC, N_PAD, DH), jnp.float32),
            jax.ShapeDtypeStruct((NC, N_PAD, L), jnp.float32),
        ),
        mesh=mesh,
        compiler_params=pltpu.CompilerParams(
            needs_layout_passes=False, use_tc_tiling_on_sc=False),
        scratch_types=[
            pltpu.VMEM_SHARED((N_PAD, DH), jnp.float32),
            pltpu.VMEM_SHARED((N_PAD, L), jnp.float32),
            pltpu.VMEM((N_PAD // D, D), jnp.float32),
            pltpu.VMEM((N_PAD // D, D), jnp.float32),
            pltpu.VMEM((NB, K), jnp.int32),
            pltpu.VMEM((NB, K), jnp.int32),
            pltpu.VMEM((NB, K, DH), jnp.float32),
            pltpu.VMEM((NB, K, L), jnp.float32),
            pltpu.VMEM((NB, K), jnp.float32),
            pltpu.VMEM((L,), jnp.float32),
            pltpu.SemaphoreType.DMA((NB,)),
            pltpu.SemaphoreType.DMA((NB,)),
            pltpu.SemaphoreType.DMA((NB,)),
            pltpu.SemaphoreType.DMA((NB,)),
        ],
    )
    def sc_edge(h_hbm, asrc_hbm, adst_hbm, src_hbm, dst_hbm, shift_hbm,
                numer_hbm, den_hbm,
                nacc, dacc, asrc_v, adst_v, src_v, dst_v,
                raw_v, wrow_v, w_v, shift_v, sem_i, sem_g, sem_s, sem_d):
        c = lax.axis_index("c")
        s = lax.axis_index("s")

        def zrow(r, carry):
            for j in range(DH // L):
                raw_v[0, r, pl.ds(j * L, L)] = jnp.zeros((L,), jnp.float32)
            for b in range(NB):
                wrow_v[b, r, pl.ds(0, L)] = jnp.zeros((L,), jnp.float32)
            return carry
        lax.fori_loop(0, K, zrow, 0)

        def zacc(t, carry):
            base = s * rows_per_sub + t * K
            pltpu.sync_copy(raw_v.at[0], nacc.at[pl.ds(base, K)])
            pltpu.sync_copy(wrow_v.at[0], dacc.at[pl.ds(base, K)])
            return carry
        lax.fori_loop(0, rows_per_sub // K, zacc, 0)

        pltpu.sync_copy(asrc_hbm, asrc_v)
        pltpu.sync_copy(adst_hbm, adst_v)
        pltpu.sync_copy(shift_hbm, shift_v)

        plsc.subcore_barrier()

        lane = lax.iota(jnp.int32, L)
        zero_lane = jnp.zeros((L,), jnp.int32)
        low_mask = jnp.full((L,), D - 1, jnp.int32)

        def issue_idx(t, b):
            pltpu.async_copy(src_hbm.at[s, t], src_v.at[b], sem_i.at[b])
            pltpu.async_copy(dst_hbm.at[s, t], dst_v.at[b], sem_i.at[b])

        def wait_idx(t, b):
            pltpu.make_async_copy(src_hbm.at[s, t], src_v.at[b], sem_i.at[b]).wait()
            pltpu.make_async_copy(dst_hbm.at[s, t], dst_v.at[b], sem_i.at[b]).wait()

        def issue_gather(b):
            pltpu.async_copy(h_hbm.at[c].at[src_v.at[b]], raw_v.at[b],
                             sem_g.at[b])

        def wait_gather(b):
            pltpu.make_async_copy(h_hbm.at[c].at[src_v.at[b]], raw_v.at[b],
                                  sem_g.at[b]).wait()

        def drain_nacc(b):
            pltpu.make_async_copy(raw_v.at[b], nacc.at[dst_v.at[b]],
                                  sem_s.at[b]).wait()

        def drain_wrow(b):
            pltpu.make_async_copy(wrow_v.at[b], dacc.at[dst_v.at[b]],
                                  sem_d.at[b]).wait()

        half = chunks // 2

        def denom_mine(tc):
            return (tc < half) == (c == 0)

        issue_idx(0, 0)
        issue_idx(1, 1)
        wait_idx(0, 0)
        issue_gather(0)

        def chunk_body(t, carry):
            @pl.when(t + 2 < chunks)
            def _():
                @pl.when(t + 2 >= NB)
                def _():
                    drain_nacc((t + 2) % NB)

                @pl.when((t + 2 >= NB) & denom_mine(t + 2 - NB))
                def _():
                    drain_wrow((t + 2) % NB)
                issue_idx(t + 2, (t + 2) % NB)

            @pl.when(t + 1 < chunks)
            def _():
                b1 = (t + 1) % NB
                wait_idx(t + 1, b1)
                issue_gather(b1)

            b = t % NB

            def wgrp(g, carry2):
                si = src_v[b, pl.ds(g * L, L)]
                di = dst_v[b, pl.ds(g * L, L)]
                a_s = plsc.load_gather(
                    asrc_v,
                    [lax.shift_right_logical(si, 7), lax.bitwise_and(si, low_mask)])
                a_d = plsc.load_gather(
                    adst_v,
                    [lax.shift_right_logical(di, 7), lax.bitwise_and(di, low_mask)])
                e = a_s + a_d
                lr = jnp.where(e >= 0.0, e, e * jnp.float32(0.2))
                w = jnp.exp(lr - shift_v[...])
                w_v[b, pl.ds(g * L, L)] = w
                plsc.store_scatter(wrow_v.at[b], [g * L + lane, zero_lane], w)
                return carry2
            lax.fori_loop(0, K // L, wgrp, 0)

            wait_gather(b)

            def erow(g, carry2):
                wv = w_v[b, pl.ds(g * L, L)]
                for ee in range(L):
                    r = g * L + ee
                    ws = wv[ee]
                    for j in range(DH // L):
                        raw_v[b, r, pl.ds(j * L, L)] = (
                            raw_v[b, r, pl.ds(j * L, L)] * ws)
                return carry2
            lax.fori_loop(0, K // L, erow, 0)

            pltpu.async_copy(raw_v.at[b], nacc.at[dst_v.at[b]], sem_s.at[b],
                             add=True)

            @pl.when(denom_mine(t))
            def _():
                pltpu.async_copy(wrow_v.at[b], dacc.at[dst_v.at[b]],
                                 sem_d.at[b], add=True)
            return carry
        lax.fori_loop(0, chunks, chunk_body, 0, unroll=2)

        for tc in range(chunks - NB, chunks):
            drain_nacc(tc % NB)

            @pl.when(denom_mine(tc))
            def _(tc=tc):
                drain_wrow(tc % NB)

        plsc.subcore_barrier()

        base = s * rows_per_sub
        pltpu.sync_copy(nacc.at[pl.ds(base, rows_per_sub)],
                        numer_hbm.at[c, pl.ds(base, rows_per_sub), :])
        pltpu.sync_copy(dacc.at[pl.ds(base, rows_per_sub)],
                        den_hbm.at[c, pl.ds(base, rows_per_sub), :])

    return sc_edge


def kernel(x, edge_index, W1, att_src1, att_dst1, b1,
           W2, att_src2, att_dst2, b2, W_lin, b_lin):
    n_real, d_in = x.shape
    n_cls = W_lin.shape[1]
    e_raw = edge_index.shape[1] + n_real
    e_pad = ((e_raw + NS * K - 1) // (NS * K)) * (NS * K)
    chunks = e_pad // (NS * K)

    loops = jnp.arange(n_real, dtype=jnp.int32)
    pad = jnp.full((e_pad - e_raw,), n_real, jnp.int32)
    src3 = jnp.concatenate([edge_index[0], loops, pad]).reshape(NS, chunks, K)
    dst3 = jnp.concatenate([edge_index[1], loops, pad]).reshape(NS, chunks, K)

    xp = jnp.zeros((N_PAD, d_in), jnp.float32).at[:n_real].set(x)

    sc_edge = _make_sc_edge(chunks)

    h1, as1, ad1, shift1 = _tc_prep(
        xp, W1, att_src1.reshape(1, D), att_dst1.reshape(1, D), n_real)
    num1, den1 = sc_edge(h1, as1.reshape(N_PAD // D, D), ad1.reshape(N_PAD // D, D),
                         src3, dst3, shift1[0, :L])

    h2, as2, ad2, shift2 = _tc_mid(
        num1, den1, b1.reshape(1, D), W2,
        att_src2.reshape(1, D), att_dst2.reshape(1, D), n_real)
    num2, den2 = sc_edge(h2, as2.reshape(N_PAD // D, D), ad2.reshape(N_PAD // D, D),
                         src3, dst3, shift2[0, :L])

    wl = jnp.zeros((D, 128), jnp.float32).at[:, :n_cls].set(W_lin)
    bl = jnp.zeros((1, 128), jnp.float32).at[0, :n_cls].set(b_lin)
    out = _tc_final(num2, den2, b2.reshape(1, D), wl, bl, n_real)
    return out[:n_real, :n_cls]


# speedup vs baseline: 1.0250x; 1.0057x over previous
import functools

import jax
import jax.numpy as jnp
from jax import lax
from jax.experimental import pallas as pl
from jax.experimental.pallas import tpu as pltpu
from jax.experimental.pallas import tpu_sc as plsc

D = 128
DH = 64
L = 16
NC = 2
NS = 16
K = 128
NB = 4
RB = 256
N_PAD = 10240


def _dense_tail(i, xin, w_ref, as_ref, ad_ref, h_ref, a_s_ref, a_d_ref,
                shift_ref, mx_ref, n_real):
    h = jnp.dot(xin, w_ref[...], preferred_element_type=jnp.float32)
    h_ref[...] = jnp.stack([h[:, :DH], h[:, DH:]])
    a_s = jnp.sum(h * as_ref[...], axis=1, keepdims=True)
    a_d = jnp.sum(h * ad_ref[...], axis=1, keepdims=True)
    rows = i * RB + lax.broadcasted_iota(jnp.int32, (RB, 1), 0)
    valid = rows < n_real
    neg_inf = jnp.float32(-jnp.inf)
    a_s = jnp.where(valid, a_s, neg_inf)
    a_d = jnp.where(valid, a_d, neg_inf)
    a_s_ref[...] = a_s
    a_d_ref[...] = a_d
    bs = jnp.max(a_s)
    bd = jnp.max(a_d)

    @pl.when(i == 0)
    def _():
        mx_ref[0] = bs
        mx_ref[1] = bd

    @pl.when(i > 0)
    def _():
        mx_ref[0] = jnp.maximum(mx_ref[0], bs)
        mx_ref[1] = jnp.maximum(mx_ref[1], bd)

    shift_ref[...] = jnp.full(
        (8, 128), jnp.maximum(mx_ref[0] + mx_ref[1], 0.0), jnp.float32)


def _prep_body(x_ref, w_ref, as_ref, ad_ref,
               h_ref, a_s_ref, a_d_ref, shift_ref, mx_ref, *, n_real):
    i = pl.program_id(0)
    _dense_tail(i, x_ref[...], w_ref, as_ref, ad_ref,
                h_ref, a_s_ref, a_d_ref, shift_ref, mx_ref, n_real)


def _gat_out_block(i, n_ref, d_ref, b_ref, n_real):
    numer = jnp.concatenate([n_ref[0], n_ref[1]], axis=-1)
    den = d_ref[0, :, 0:1] + d_ref[1, :, 0:1]
    xin = numer / (den + 1e-16) + b_ref[...]
    xin = jnp.where(xin > 0, xin, jnp.exp(jnp.minimum(xin, 0.0)) - 1.0)
    rows = i * RB + lax.broadcasted_iota(jnp.int32, (RB, 1), 0)
    return jnp.where(rows < n_real, xin, 0.0)


def _mid_body(n_ref, d_ref, b_ref, w_ref, as_ref, ad_ref,
              h_ref, a_s_ref, a_d_ref, shift_ref, mx_ref, *, n_real):
    i = pl.program_id(0)
    xin = _gat_out_block(i, n_ref, d_ref, b_ref, n_real)
    _dense_tail(i, xin, w_ref, as_ref, ad_ref,
                h_ref, a_s_ref, a_d_ref, shift_ref, mx_ref, n_real)


def _final_body(n_ref, d_ref, b_ref, w_ref, bl_ref, out_ref, *, n_real):
    i = pl.program_id(0)
    xin = _gat_out_block(i, n_ref, d_ref, b_ref, n_real)
    out_ref[...] = (
        jnp.dot(xin, w_ref[...], preferred_element_type=jnp.float32)
        + bl_ref[...])


_DENSE_OUT = [
    jax.ShapeDtypeStruct((NC, N_PAD, DH), jnp.float32),
    jax.ShapeDtypeStruct((N_PAD, 1), jnp.float32),
    jax.ShapeDtypeStruct((N_PAD, 1), jnp.float32),
    jax.ShapeDtypeStruct((8, 128), jnp.float32),
]
_DENSE_OUT_SPECS = [
    pl.BlockSpec((NC, RB, DH), lambda i: (0, i, 0)),
    pl.BlockSpec((RB, 1), lambda i: (i, 0)),
    pl.BlockSpec((RB, 1), lambda i: (i, 0)),
    pl.BlockSpec((8, 128), lambda i: (0, 0)),
]


def _tc_prep(xp, w, att_s, att_d, n_real):
    grid = N_PAD // RB
    return pl.pallas_call(
        functools.partial(_prep_body, n_real=n_real),
        grid=(grid,),
        in_specs=[
            pl.BlockSpec((RB, D), lambda i: (i, 0)),
            pl.BlockSpec((D, D), lambda i: (0, 0)),
            pl.BlockSpec((1, D), lambda i: (0, 0)),
            pl.BlockSpec((1, D), lambda i: (0, 0)),
        ],
        out_specs=_DENSE_OUT_SPECS,
        out_shape=_DENSE_OUT,
        scratch_shapes=[pltpu.SMEM((2,), jnp.float32)],
    )(xp, w, att_s, att_d)


def _tc_mid(num, den0, b, w, att_s, att_d, n_real):
    grid = N_PAD // RB
    return pl.pallas_call(
        functools.partial(_mid_body, n_real=n_real),
        grid=(grid,),
        in_specs=[
            pl.BlockSpec((NC, RB, DH), lambda i: (0, i, 0)),
            pl.BlockSpec((NC, RB, L), lambda i: (0, i, 0)),
            pl.BlockSpec((1, D), lambda i: (0, 0)),
            pl.BlockSpec((D, D), lambda i: (0, 0)),
            pl.BlockSpec((1, D), lambda i: (0, 0)),
            pl.BlockSpec((1, D), lambda i: (0, 0)),
        ],
        out_specs=_DENSE_OUT_SPECS,
        out_shape=_DENSE_OUT,
        scratch_shapes=[pltpu.SMEM((2,), jnp.float32)],
    )(num, den0, b, w, att_s, att_d)


def _tc_final(num, den0, b, w, bl, n_real):
    grid = N_PAD // RB
    return pl.pallas_call(
        functools.partial(_final_body, n_real=n_real),
        grid=(grid,),
        in_specs=[
            pl.BlockSpec((NC, RB, DH), lambda i: (0, i, 0)),
            pl.BlockSpec((NC, RB, L), lambda i: (0, i, 0)),
            pl.BlockSpec((1, D), lambda i: (0, 0)),
            pl.BlockSpec((D, D), lambda i: (0, 0)),
            pl.BlockSpec((1, D), lambda i: (0, 0)),
        ],
        out_specs=pl.BlockSpec((RB, D), lambda i: (i, 0)),
        out_shape=jax.ShapeDtypeStruct((N_PAD, D), jnp.float32),
    )(num, den0, b, w, bl)


def _make_sc_edge(chunks):
    mesh = plsc.VectorSubcoreMesh(core_axis_name="c", subcore_axis_name="s")
    rows_per_sub = N_PAD // NS

    @functools.partial(
        pl.kernel,
        out_type=(
            jax.ShapeDtypeStruct((NC, N_PAD, DH), jnp.float32),
            jax.ShapeDtypeStruct((NC, N_PAD, L), jnp.float32),
        ),
        mesh=mesh,
        compiler_params=pltpu.CompilerParams(
            needs_layout_passes=False, use_tc_tiling_on_sc=False,
            disable_bounds_checks=True),
        scratch_types=[
            pltpu.VMEM_SHARED((N_PAD, DH), jnp.float32),
            pltpu.VMEM_SHARED((N_PAD, L), jnp.float32),
            pltpu.VMEM((N_PAD // D, D), jnp.float32),
            pltpu.VMEM((N_PAD // D, D), jnp.float32),
            pltpu.VMEM((NB, K), jnp.int32),
            pltpu.VMEM((NB, K), jnp.int32),
            pltpu.VMEM((NB, K, DH), jnp.float32),
            pltpu.VMEM((NB, K, L), jnp.float32),
            pltpu.VMEM((NB, K), jnp.float32),
            pltpu.VMEM((L,), jnp.float32),
            pltpu.SemaphoreType.DMA((NB,)),
            pltpu.SemaphoreType.DMA((NB,)),
            pltpu.SemaphoreType.DMA((NB,)),
            pltpu.SemaphoreType.DMA((NB,)),
        ],
    )
    def sc_edge(h_hbm, asrc_hbm, adst_hbm, src_hbm, dst_hbm, shift_hbm,
                numer_hbm, den_hbm,
                nacc, dacc, asrc_v, adst_v, src_v, dst_v,
                raw_v, wrow_v, w_v, shift_v, sem_i, sem_g, sem_s, sem_d):
        c = lax.axis_index("c")
        s = lax.axis_index("s")

        def issue_idx(t, b):
            pltpu.async_copy(src_hbm.at[s, t], src_v.at[b], sem_i.at[b])
            pltpu.async_copy(dst_hbm.at[s, t], dst_v.at[b], sem_i.at[b])

        def wait_idx(t, b):
            pltpu.make_async_copy(src_hbm.at[s, t], src_v.at[b], sem_i.at[b]).wait()
            pltpu.make_async_copy(dst_hbm.at[s, t], dst_v.at[b], sem_i.at[b]).wait()

        issue_idx(0, 0)
        issue_idx(1, 1)
        pltpu.sync_copy(asrc_hbm, asrc_v)
        pltpu.sync_copy(adst_hbm, adst_v)
        pltpu.sync_copy(shift_hbm, shift_v)

        def zrow(r, carry):
            for j in range(DH // L):
                raw_v[0, r, pl.ds(j * L, L)] = jnp.zeros((L,), jnp.float32)
            for b in range(NB):
                wrow_v[b, r, pl.ds(0, L)] = jnp.zeros((L,), jnp.float32)
            return carry
        lax.fori_loop(0, K, zrow, 0)

        def zacc(t, carry):
            base = s * rows_per_sub + t * K
            pltpu.sync_copy(raw_v.at[0], nacc.at[pl.ds(base, K)])
            pltpu.sync_copy(wrow_v.at[0], dacc.at[pl.ds(base, K)])
            return carry
        lax.fori_loop(0, rows_per_sub // K, zacc, 0)

        lane = lax.iota(jnp.int32, L)
        zero_lane = jnp.zeros((L,), jnp.int32)
        low_mask = jnp.full((L,), D - 1, jnp.int32)

        def issue_gather(b):
            pltpu.async_copy(h_hbm.at[c].at[src_v.at[b]], raw_v.at[b],
                             sem_g.at[b])

        def wait_gather(b):
            pltpu.make_async_copy(h_hbm.at[c].at[src_v.at[b]], raw_v.at[b],
                                  sem_g.at[b]).wait()

        def drain_nacc(b):
            pltpu.make_async_copy(raw_v.at[b], nacc.at[dst_v.at[b]],
                                  sem_s.at[b]).wait()

        def drain_wrow(b):
            pltpu.make_async_copy(wrow_v.at[b], dacc.at[dst_v.at[b]],
                                  sem_d.at[b]).wait()

        half = chunks // 2

        def denom_mine(tc):
            return (tc < half) == (c == 0)

        wait_idx(0, 0)
        issue_gather(0)

        plsc.subcore_barrier()

        def chunk_body(t, carry):
            @pl.when(t + 2 < chunks)
            def _():
                @pl.when(t + 2 >= NB)
                def _():
                    drain_nacc((t + 2) % NB)

                @pl.when((t + 2 >= NB) & denom_mine(t + 2 - NB))
                def _():
                    drain_wrow((t + 2) % NB)
                issue_idx(t + 2, (t + 2) % NB)

            @pl.when(t + 1 < chunks)
            def _():
                b1 = (t + 1) % NB
                wait_idx(t + 1, b1)
                issue_gather(b1)

            b = t % NB

            def wgrp(g, carry2):
                si = src_v[b, pl.ds(g * L, L)]
                di = dst_v[b, pl.ds(g * L, L)]
                a_s = plsc.load_gather(
                    asrc_v,
                    [lax.shift_right_logical(si, 7), lax.bitwise_and(si, low_mask)])
                a_d = plsc.load_gather(
                    adst_v,
                    [lax.shift_right_logical(di, 7), lax.bitwise_and(di, low_mask)])
                e = a_s + a_d
                lr = jnp.where(e >= 0.0, e, e * jnp.float32(0.2))
                w = jnp.exp(lr - shift_v[...])
                w_v[b, pl.ds(g * L, L)] = w
                plsc.store_scatter(wrow_v.at[b], [g * L + lane, zero_lane], w)
                return carry2
            lax.fori_loop(0, K // L, wgrp, 0)

            wait_gather(b)

            def erow(g, carry2):
                wv = w_v[b, pl.ds(g * L, L)]
                for ee in range(L):
                    r = g * L + ee
                    ws = wv[ee]
                    for j in range(DH // L):
                        raw_v[b, r, pl.ds(j * L, L)] = (
                            raw_v[b, r, pl.ds(j * L, L)] * ws)
                return carry2
            lax.fori_loop(0, K // L, erow, 0)

            pltpu.async_copy(raw_v.at[b], nacc.at[dst_v.at[b]], sem_s.at[b],
                             add=True)

            @pl.when(denom_mine(t))
            def _():
                pltpu.async_copy(wrow_v.at[b], dacc.at[dst_v.at[b]],
                                 sem_d.at[b], add=True)
            return carry
        lax.fori_loop(0, chunks, chunk_body, 0, unroll=2)

        for tc in range(chunks - NB, chunks):
            drain_nacc(tc % NB)

            @pl.when(denom_mine(tc))
            def _(tc=tc):
                drain_wrow(tc % NB)

        plsc.subcore_barrier()

        base = s * rows_per_sub
        pltpu.sync_copy(nacc.at[pl.ds(base, rows_per_sub)],
                        numer_hbm.at[c, pl.ds(base, rows_per_sub), :])
        pltpu.sync_copy(dacc.at[pl.ds(base, rows_per_sub)],
                        den_hbm.at[c, pl.ds(base, rows_per_sub), :])

    return sc_edge


def kernel(x, edge_index, W1, att_src1, att_dst1, b1,
           W2, att_src2, att_dst2, b2, W_lin, b_lin):
    n_real, d_in = x.shape
    n_cls = W_lin.shape[1]
    e_raw = edge_index.shape[1] + n_real
    e_pad = ((e_raw + NS * K - 1) // (NS * K)) * (NS * K)
    chunks = e_pad // (NS * K)

    loops = jnp.arange(n_real, dtype=jnp.int32)
    pad = jnp.full((e_pad - e_raw,), n_real, jnp.int32)
    src3 = jnp.concatenate([edge_index[0], loops, pad]).reshape(NS, chunks, K)
    dst3 = jnp.concatenate([edge_index[1], loops, pad]).reshape(NS, chunks, K)

    xp = jnp.zeros((N_PAD, d_in), jnp.float32).at[:n_real].set(x)

    sc_edge = _make_sc_edge(chunks)

    h1, as1, ad1, shift1 = _tc_prep(
        xp, W1, att_src1.reshape(1, D), att_dst1.reshape(1, D), n_real)
    num1, den1 = sc_edge(h1, as1.reshape(N_PAD // D, D), ad1.reshape(N_PAD // D, D),
                         src3, dst3, shift1[0, :L])

    h2, as2, ad2, shift2 = _tc_mid(
        num1, den1, b1.reshape(1, D), W2,
        att_src2.reshape(1, D), att_dst2.reshape(1, D), n_real)
    num2, den2 = sc_edge(h2, as2.reshape(N_PAD // D, D), ad2.reshape(N_PAD // D, D),
                         src3, dst3, shift2[0, :L])

    wl = jnp.zeros((D, 128), jnp.float32).at[:, :n_cls].set(W_lin)
    bl = jnp.zeros((1, 128), jnp.float32).at[0, :n_cls].set(b_lin)
    out = _tc_final(num2, den2, b2.reshape(1, D), wl, bl, n_real)
    return out[:n_real, :n_cls]
